# baseline (speedup 1.0000x reference)
import functools

import jax
import jax.numpy as jnp
from jax import lax
from jax.experimental import pallas as pl
from jax.experimental.pallas import tpu as pltpu
from jax.experimental.pallas import tpu_sc as plsc

N = 10000
E = 320000
D = 128
G = 128
L = 3

NC = 2
NS = 16

K = 128
CH0 = 102
CH1 = 56
EP = NS * (CH0 + CH1) * K

NPAD = 16 * 632
DUMMY = N + 8
ZR = 632
OUTR = 624
OUT_TAIL = N - NS * OUTR


def _sc_agg_body(z_hbm, src_hbm, dst_hbm, out_hbm, idx_s, idx_d, rows, acc, sem):
    c = lax.axis_index("c")
    s = lax.axis_index("s")

    def zero_body(i, _):
        rows[i // 8, pl.ds((i % 8) * 16, 16)] = jnp.zeros((16,), jnp.float32)
        return 0

    lax.fori_loop(0, K * 8, zero_body, 0)
    zbase = s * ZR
    for t in range(ZR // K):
        pltpu.sync_copy(rows, acc.at[pl.ds(zbase + t * K, K)])
    rem = ZR % K
    if rem:
        pltpu.sync_copy(rows.at[pl.ds(0, rem)], acc.at[pl.ds(zbase + (ZR // K) * K, rem)])
    plsc.subcore_barrier()

    wbase = jnp.where(c == 0, s * (CH0 * K), NS * (CH0 * K) + s * (CH1 * K))
    nchunks = jnp.where(c == 0, CH0, CH1)

    def chunk_body(k, _):
        off = wbase + k * K
        pltpu.sync_copy(src_hbm.at[pl.ds(off, K)], idx_s)
        pltpu.sync_copy(dst_hbm.at[pl.ds(off, K)], idx_d)
        pltpu.async_copy(z_hbm.at[idx_s], rows, sem).wait()
        pltpu.sync_copy(rows, acc.at[idx_d], add=True)
        return 0

    lax.fori_loop(0, nchunks, chunk_body, 0)
    plsc.subcore_barrier()

    obase = s * OUTR
    pltpu.sync_copy(acc.at[pl.ds(obase, OUTR)], out_hbm.at[c, pl.ds(obase, OUTR)])

    @pl.when(s == 0)
    def _():
        pltpu.sync_copy(acc.at[pl.ds(NS * OUTR, OUT_TAIL)],
                        out_hbm.at[c, pl.ds(NS * OUTR, OUT_TAIL)])


@functools.cache
def _sc_agg():
    return pl.kernel(
        _sc_agg_body,
        out_type=jax.ShapeDtypeStruct((NC, N, D), jnp.float32),
        mesh=plsc.VectorSubcoreMesh(core_axis_name="c", subcore_axis_name="s"),
        scratch_types=[
            pltpu.VMEM((K,), jnp.int32),
            pltpu.VMEM((K,), jnp.int32),
            pltpu.VMEM((K, D), jnp.float32),
            pltpu.VMEM_SHARED((NPAD, D), jnp.float32),
            pltpu.SemaphoreType.DMA,
        ],
    )


def _tc_layer_body(z_ref, agg_ref, w1_ref, b1_ref, w2_ref, b2_ref, gam_ref,
                   bet_ref, batch_ref, zout_ref, g_ref):
    z = z_ref[...]
    h = z + agg_ref[0] + agg_ref[1]
    h = jnp.dot(h, w1_ref[...], preferred_element_type=jnp.float32) + b1_ref[...][None, :]
    h = jnp.maximum(h, 0.0)
    h = jnp.dot(h, w2_ref[...], preferred_element_type=jnp.float32) + b2_ref[...][None, :]
    zr = jnp.maximum(h, 0.0)
    mu = jnp.sum(zr, axis=0) * (1.0 / N)
    d = zr - mu[None, :]
    var = jnp.sum(d * d, axis=0) * (1.0 / N)
    zn = d * lax.rsqrt(var + 1e-5)[None, :] * gam_ref[...][None, :] + bet_ref[...][None, :]
    zout_ref[...] = zn
    ids = batch_ref[...]
    iota = lax.broadcasted_iota(jnp.int32, (G, N), 0)
    oh = jnp.where(iota == ids[None, :], 1.0, 0.0)
    g_ref[...] = jnp.dot(oh, zn, preferred_element_type=jnp.float32,
                         precision=lax.Precision.HIGHEST)


_tc_layer = pl.pallas_call(
    _tc_layer_body,
    out_shape=[
        jax.ShapeDtypeStruct((N, D), jnp.float32),
        jax.ShapeDtypeStruct((G, D), jnp.float32),
    ],
)


def kernel(x, edge_index, batch, W1, b1, W2, b2, gamma, beta):
    pad = EP - E
    srcp = jnp.concatenate([edge_index[0], jnp.zeros((pad,), jnp.int32)])
    dstp = jnp.concatenate([edge_index[1], jnp.full((pad,), DUMMY, jnp.int32)])
    z = x
    gs = []
    for i in range(L):
        agg = _sc_agg()(z, srcp, dstp)
        z, g = _tc_layer(z, agg, W1[i], b1[i], W2[i], b2[i], gamma, beta, batch)
        gs.append(g)
    return jnp.concatenate(gs, axis=1)


# speedup vs baseline: 1.0054x; 1.0054x over previous
import functools

import jax
import jax.numpy as jnp
from jax import lax
from jax.experimental import pallas as pl
from jax.experimental.pallas import tpu as pltpu
from jax.experimental.pallas import tpu_sc as plsc

N = 10000
E = 320000
D = 128
G = 128
L = 3

NC = 2
NS = 16

K = 128
CH0 = 100
CH1 = 58
EP = NS * (CH0 + CH1) * K

NPAD = 16 * 632
DUMMY = N + 8
ZR = 632
OUTR = 624
OUT_TAIL = N - NS * OUTR


def _sc_agg_body(z_hbm, src_hbm, dst_hbm, out_hbm, idx_s, idx_d, rows, acc, sem):
    c = lax.axis_index("c")
    s = lax.axis_index("s")

    def zero_body(i, _):
        rows[i // 8, pl.ds((i % 8) * 16, 16)] = jnp.zeros((16,), jnp.float32)
        return 0

    lax.fori_loop(0, K * 8, zero_body, 0)
    zbase = s * ZR
    for t in range(ZR // K):
        pltpu.sync_copy(rows, acc.at[pl.ds(zbase + t * K, K)])
    rem = ZR % K
    if rem:
        pltpu.sync_copy(rows.at[pl.ds(0, rem)], acc.at[pl.ds(zbase + (ZR // K) * K, rem)])
    plsc.subcore_barrier()

    wbase = jnp.where(c == 0, s * (CH0 * K), NS * (CH0 * K) + s * (CH1 * K))
    nchunks = jnp.where(c == 0, CH0, CH1)

    def chunk_body(k, _):
        off = wbase + k * K
        pltpu.sync_copy(src_hbm.at[pl.ds(off, K)], idx_s)
        pltpu.sync_copy(dst_hbm.at[pl.ds(off, K)], idx_d)
        pltpu.async_copy(z_hbm.at[idx_s], rows, sem).wait()
        pltpu.sync_copy(rows, acc.at[idx_d], add=True)
        return 0

    lax.fori_loop(0, nchunks, chunk_body, 0)
    plsc.subcore_barrier()

    obase = s * OUTR
    pltpu.sync_copy(acc.at[pl.ds(obase, OUTR)], out_hbm.at[c, pl.ds(obase, OUTR)])

    @pl.when(s == 0)
    def _():
        pltpu.sync_copy(acc.at[pl.ds(NS * OUTR, OUT_TAIL)],
                        out_hbm.at[c, pl.ds(NS * OUTR, OUT_TAIL)])


@functools.cache
def _sc_agg():
    return pl.kernel(
        _sc_agg_body,
        out_type=jax.ShapeDtypeStruct((NC, N, D), jnp.float32),
        mesh=plsc.VectorSubcoreMesh(core_axis_name="c", subcore_axis_name="s"),
        scratch_types=[
            pltpu.VMEM((K,), jnp.int32),
            pltpu.VMEM((K,), jnp.int32),
            pltpu.VMEM((K, D), jnp.float32),
            pltpu.VMEM_SHARED((NPAD, D), jnp.float32),
            pltpu.SemaphoreType.DMA,
        ],
    )


def _tc_layer_body(z_ref, agg_ref, w1_ref, b1_ref, w2_ref, b2_ref, gam_ref,
                   bet_ref, batch_ref, zout_ref, g_ref):
    z = z_ref[...]
    h = z + agg_ref[0] + agg_ref[1]
    h = jnp.dot(h, w1_ref[...], preferred_element_type=jnp.float32) + b1_ref[...][None, :]
    h = jnp.maximum(h, 0.0)
    h = jnp.dot(h, w2_ref[...], preferred_element_type=jnp.float32) + b2_ref[...][None, :]
    zr = jnp.maximum(h, 0.0)
    mu = jnp.sum(zr, axis=0) * (1.0 / N)
    d = zr - mu[None, :]
    var = jnp.sum(d * d, axis=0) * (1.0 / N)
    zn = d * lax.rsqrt(var + 1e-5)[None, :] * gam_ref[...][None, :] + bet_ref[...][None, :]
    zout_ref[...] = zn
    ids = batch_ref[...]
    iota = lax.broadcasted_iota(jnp.int32, (G, N), 0)
    oh = jnp.where(iota == ids[None, :], 1.0, 0.0)
    g_ref[...] = jnp.dot(oh, zn, preferred_element_type=jnp.float32,
                         precision=lax.Precision.HIGHEST)


_tc_layer = pl.pallas_call(
    _tc_layer_body,
    out_shape=[
        jax.ShapeDtypeStruct((N, D), jnp.float32),
        jax.ShapeDtypeStruct((G, D), jnp.float32),
    ],
)


def kernel(x, edge_index, batch, W1, b1, W2, b2, gamma, beta):
    pad = EP - E
    srcp = jnp.concatenate([edge_index[0], jnp.zeros((pad,), jnp.int32)])
    dstp = jnp.concatenate([edge_index[1], jnp.full((pad,), DUMMY, jnp.int32)])
    z = x
    gs = []
    for i in range(L):
        agg = _sc_agg()(z, srcp, dstp)
        z, g = _tc_layer(z, agg, W1[i], b1[i], W2[i], b2[i], gamma, beta, batch)
        gs.append(g)
    return jnp.concatenate(gs, axis=1)
